# Initial kernel scaffold; baseline (speedup 1.0000x reference)
import jax
import jax.numpy as jnp
from jax.experimental import pallas as pl


def kernel(pl_name_src, track_name_pl, track_danceability_pl, track_energy_pl, track_key_pl, track_loudness_pl, track_mode_pl, track_speechiness_pl, track_acousticness_pl, track_instrumentalness_pl, track_liveness_pl, track_valence_pl, track_tempo_pl, time_signature_pl, track_name_can, artist_genres_can, track_danceability_can, track_energy_can, track_key_can, track_loudness_can, track_mode_can, track_speechiness_can, T_pl_name_src, T_track_name_pl, T_track_danceability_pl, T_track_energy_pl, T_track_key_pl, T_track_loudness_pl, T_track_mode_pl, T_track_speechiness_pl, T_track_acousticness_pl, T_track_instrumentalness_pl, T_track_liveness_pl, T_track_valence_pl, T_track_tempo_pl, T_time_signature_pl, T_track_name_can, T_artist_genres_can, T_track_danceability_can, T_track_energy_can, T_track_key_can, T_track_loudness_can, T_track_mode_can, T_track_speechiness_can):
    raise NotImplementedError("write your pallas kernel here")


# speedup vs baseline: 6.1179x; 6.1179x over previous
import functools

import jax
import jax.numpy as jnp
from jax import lax
from jax.experimental import pallas as pl
from jax.experimental.pallas import tpu as pltpu
from jax.experimental.pallas import tpu_sc as plsc

B = 1024
D = 128
NW = 32
NB = B // NW

PL_VOCABS = (21, 21, 13, 21, 4, 21, 21, 21, 21, 21, 21, 7)
CS_VOCABS = (21, 21, 13, 21, 4, 21)
TBL_ROWS = sum(PL_VOCABS)


def _sc_body(refs):
    (ps_i, tn_i, tc_i, ag_i, pl_is, cs_is,
     ps_T, tn_T, tc_T, ag_T, pl_Ts, cs_Ts,
     out,
     idx16, idx80, idx20, idx1d,
     rbig, rsm, rcs, tblv, t0v, outv,
     sem, semt) = refs

    cid = lax.axis_index("c")
    sid = lax.axis_index("s")
    wid = sid * 2 + cid
    base = wid * NB

    zero16 = jnp.zeros((16,), jnp.float32)

    def accum_rows(buf, n_rows, nv):
        def s_body(s, acc):
            return tuple(acc[c] + buf[s, pl.ds(c * 16, 16)] for c in range(nv))
        return lax.fori_loop(0, n_rows, s_body, (zero16,) * nv)

    def store_row(b, vecs, scale):
        for c in range(len(vecs)):
            outv[b, pl.ds(c * 16, 16)] = vecs[c] * scale

    def flush(col):
        pltpu.sync_copy(outv, out.at[pl.ds(base, NB), pl.ds(col, 128)])

    def text128(idx_hbm, T_hbm, S, col, masked=False):
        pltpu.sync_copy(idx_hbm.at[pl.ds(base * S, NB * S)], idx16)

        def b_body(b, _):
            pltpu.async_copy(T_hbm.at[idx16.at[pl.ds(b * S, S)]], rsm, sem).wait()
            acc = accum_rows(rsm, S, 8)
            if masked:
                va = idx16[pl.ds(b * 16, 16)]
                nz = va[0] * 0
                for s in range(16):
                    nz = nz + jnp.where(va[s] == 0, 1, 0)
                nz = nz.astype(jnp.float32)
                nzv = lax.broadcast_in_dim(nz, (16,), ())
                inv = 1.0 / jnp.maximum(jnp.float32(S) - nzv, 1.0)
                vecs = tuple((acc[c] - nzv * t0v[0, pl.ds(c * 16, 16)]) * inv
                             for c in range(8))
                store_row(b, vecs, 1.0)
            else:
                store_row(b, acc, 1.0 / S)
            return 0

        lax.fori_loop(0, NB, b_body, 0)
        flush(col)

    text128(ps_i, ps_T, 16, 0)

    pltpu.sync_copy(tn_i.at[pl.ds(base * 80, NB * 80)], idx80)

    def tn_body(b, _):
        pltpu.async_copy(tn_T.at[idx80.at[pl.ds(b * 80, 80)]], rbig, sem).wait()
        acc = accum_rows(rbig, 80, 16)
        folded = tuple(acc[c] + acc[c + 8] for c in range(8))
        store_row(b, folded, 1.0 / 160.0)
        return 0

    lax.fori_loop(0, NB, tn_body, 0)
    flush(128)

    roff = 0
    for f in range(12):
        pltpu.sync_copy(pl_Ts[f], tblv.at[pl.ds(roff, PL_VOCABS[f])])
        roff += PL_VOCABS[f]

    roff = 0
    for f in range(12):
        pltpu.sync_copy(pl_is[f].at[pl.ds(base * 20, NB * 20)], idx20.at[pl.ds(0, NB * 20)])
        toff = roff

        def pb_body(b, _, toff=toff):
            def s_body(s, acc):
                row = idx20[pl.ds(b * 20 + s, 16)][0] + toff
                return tuple(acc[c] + tblv[row, pl.ds(c * 16, 16)]
                             for c in range(8))
            acc = lax.fori_loop(0, 20, s_body, (zero16,) * 8)
            store_row(b, acc, 1.0 / 20.0)
            return 0

        lax.fori_loop(0, NB, pb_body, 0)
        flush(256 + f * 128)
        roff += PL_VOCABS[f]

    text128(tc_i, tc_T, 16, 1792)
    pltpu.sync_copy(ag_T.at[pl.ds(0, 1)], t0v)
    text128(ag_i, ag_T, 16, 1920, masked=True)

    for j in range(6):
        pltpu.sync_copy(cs_is[j].at[pl.ds(base, NB)], idx1d)
        pltpu.async_copy(cs_Ts[j].at[idx1d], rcs, semt).wait()
        pltpu.sync_copy(rcs, out.at[pl.ds(base, NB), pl.ds(2048 + j * 128, 128)])


def _sc_forward(ps_i, tn_i, tc_i, ag_i, pl_is, cs_is,
                ps_T, tn_T, tc_T, ag_T, pl_Ts, cs_Ts):
    mesh = plsc.VectorSubcoreMesh(core_axis_name="c", subcore_axis_name="s",
                                  num_cores=2, num_subcores=16)
    scratch = [
        pltpu.VMEM((NB * 16,), jnp.int32),
        pltpu.VMEM((NB * 80,), jnp.int32),
        pltpu.VMEM((NB * 20 + 16,), jnp.int32),
        pltpu.VMEM((NB,), jnp.int32),
        pltpu.VMEM((80, 256), jnp.float32),
        pltpu.VMEM((16, 128), jnp.float32),
        pltpu.VMEM((NB, 128), jnp.float32),
        pltpu.VMEM((TBL_ROWS, 128), jnp.float32),
        pltpu.VMEM((1, 128), jnp.float32),
        pltpu.VMEM((NB, 128), jnp.float32),
        pltpu.SemaphoreType.DMA,
        pltpu.SemaphoreType.DMA,
    ]

    def body(*refs):
        n_pl, n_cs = 12, 6
        it = list(refs)
        ps_ir, tn_ir, tc_ir, ag_ir = it[0:4]
        pl_irs = it[4:16]
        cs_irs = it[16:22]
        ps_Tr, tn_Tr, tc_Tr, ag_Tr = it[22:26]
        pl_Trs = it[26:38]
        cs_Trs = it[38:44]
        out = it[44]
        rest = it[45:]
        _sc_body((ps_ir, tn_ir, tc_ir, ag_ir, pl_irs, cs_irs,
                  ps_Tr, tn_Tr, tc_Tr, ag_Tr, pl_Trs, cs_Trs,
                  out, *rest))

    fn = pl.kernel(body,
                   out_type=jax.ShapeDtypeStruct((B, 22 * D), jnp.float32),
                   mesh=mesh, scratch_types=scratch)
    return fn(ps_i, tn_i, tc_i, ag_i, *pl_is, *cs_is,
              ps_T, tn_T, tc_T, ag_T, *pl_Ts, *cs_Ts)


def kernel(pl_name_src, track_name_pl, track_danceability_pl, track_energy_pl,
           track_key_pl, track_loudness_pl, track_mode_pl, track_speechiness_pl,
           track_acousticness_pl, track_instrumentalness_pl, track_liveness_pl,
           track_valence_pl, track_tempo_pl, time_signature_pl, track_name_can,
           artist_genres_can, track_danceability_can, track_energy_can,
           track_key_can, track_loudness_can, track_mode_can, track_speechiness_can,
           T_pl_name_src, T_track_name_pl, T_track_danceability_pl, T_track_energy_pl,
           T_track_key_pl, T_track_loudness_pl, T_track_mode_pl, T_track_speechiness_pl,
           T_track_acousticness_pl, T_track_instrumentalness_pl, T_track_liveness_pl,
           T_track_valence_pl, T_track_tempo_pl, T_time_signature_pl, T_track_name_can,
           T_artist_genres_can, T_track_danceability_can, T_track_energy_can,
           T_track_key_can, T_track_loudness_can, T_track_mode_can, T_track_speechiness_can):
    pl_is = [track_danceability_pl, track_energy_pl, track_key_pl,
             track_loudness_pl, track_mode_pl, track_speechiness_pl,
             track_acousticness_pl, track_instrumentalness_pl,
             track_liveness_pl, track_valence_pl, track_tempo_pl,
             time_signature_pl]
    cs_is = [track_danceability_can, track_energy_can, track_key_can,
             track_loudness_can, track_mode_can, track_speechiness_can]
    pl_Ts = [T_track_danceability_pl, T_track_energy_pl, T_track_key_pl,
             T_track_loudness_pl, T_track_mode_pl, T_track_speechiness_pl,
             T_track_acousticness_pl, T_track_instrumentalness_pl,
             T_track_liveness_pl, T_track_valence_pl, T_track_tempo_pl,
             T_time_signature_pl]
    cs_Ts = [T_track_danceability_can, T_track_energy_can, T_track_key_can,
             T_track_loudness_can, T_track_mode_can, T_track_speechiness_can]
    pl_is = [jnp.ravel(x) for x in pl_is]
    return _sc_forward(jnp.ravel(pl_name_src), jnp.ravel(track_name_pl),
                       jnp.ravel(track_name_can),
                       jnp.ravel(artist_genres_can), pl_is, cs_is,
                       T_pl_name_src, T_track_name_pl, T_track_name_can,
                       T_artist_genres_can, pl_Ts, cs_Ts)


# speedup vs baseline: 8.3195x; 1.3599x over previous
import jax
import jax.numpy as jnp
from jax import lax
from jax.experimental import pallas as pl
from jax.experimental.pallas import tpu as pltpu
from jax.experimental.pallas import tpu_sc as plsc

B = 1024
D = 128
NW = 32
NB = B // NW
CH = 8

PL_VOCABS = (21, 21, 13, 21, 4, 21, 21, 21, 21, 21, 21, 7)
TBL_ROWS = sum(PL_VOCABS)


def _sc_body(refs):
    (ps_i, tn_i, tc_i, ag_i, pl_is, cs_is,
     ps_T, tn_T, tc_T, ag_T, pl_Ts, cs_Ts,
     out,
     idx16, idx80, idx20, idx1d,
     rsmA, rsmB, rbigA, rbigB, rcsA, rcsB, tblv, t0v, outv,
     semA, semB) = refs

    cid = lax.axis_index("c")
    sid = lax.axis_index("s")
    wid = sid * 2 + cid
    base = wid * NB

    zero16 = jnp.zeros((16,), jnp.float32)
    sems = (semA, semB)

    def ring(n, fire, wait, consume):
        fire(0, 0)

        def g_body(g, _):
            c0 = 2 * g
            wait(0)
            fire(c0 + 1, 1)
            consume(c0, 0)
            wait(1)

            @pl.when(c0 + 2 < n)
            def _():
                fire(c0 + 2, 0)

            consume(c0 + 1, 1)
            return 0

        lax.fori_loop(0, n // 2, g_body, 0)

    def store_row(b, vecs, scale):
        for c in range(len(vecs)):
            outv[b, pl.ds(c * 16, 16)] = vecs[c] * scale

    def flush(col):
        pltpu.sync_copy(outv, out.at[pl.ds(base, NB), pl.ds(col, 128)])

    def text128(idx_hbm, T_hbm, col, masked=False):
        S = 16
        bufs = (rsmA, rsmB)
        pltpu.sync_copy(idx_hbm.at[pl.ds(base * S, NB * S)], idx16)

        def fire(c, k):
            pltpu.async_copy(T_hbm.at[idx16.at[pl.ds(c * CH * S, CH * S)]],
                             bufs[k], sems[k])

        def wait(k):
            pltpu.make_async_copy(T_hbm.at[pl.ds(0, CH * S)],
                                  bufs[k], sems[k]).wait()

        def consume(c, k):
            buf = bufs[k]

            def bb_body(bb, _):
                b = c * CH + bb

                def s_body(s, acc):
                    return tuple(acc[j] + buf[bb * S + s, pl.ds(j * 16, 16)]
                                 for j in range(8))

                acc = lax.fori_loop(0, S, s_body, (zero16,) * 8)
                if masked:
                    va = idx16[pl.ds(b * S, 16)]
                    nz = va[0] * 0
                    for s in range(16):
                        nz = nz + jnp.where(va[s] == 0, 1, 0)
                    nzv = lax.broadcast_in_dim(nz.astype(jnp.float32), (16,), ())
                    inv = 1.0 / jnp.maximum(jnp.float32(S) - nzv, 1.0)
                    vecs = tuple((acc[j] - nzv * t0v[0, pl.ds(j * 16, 16)]) * inv
                                 for j in range(8))
                    store_row(b, vecs, 1.0)
                else:
                    store_row(b, acc, 1.0 / S)
                return 0

            lax.fori_loop(0, CH, bb_body, 0)

        ring(NB // CH, fire, wait, consume)
        flush(col)

    text128(ps_i, ps_T, 0)

    pltpu.sync_copy(tn_i.at[pl.ds(base * 80, NB * 80)], idx80)
    tn_bufs = (rbigA, rbigB)

    def tn_fire(b, k):
        pltpu.async_copy(tn_T.at[idx80.at[pl.ds(b * 80, 80)]],
                         tn_bufs[k], sems[k])

    def tn_wait(k):
        pltpu.make_async_copy(tn_T.at[pl.ds(0, 80)], tn_bufs[k], sems[k]).wait()

    def tn_consume(b, k):
        buf = tn_bufs[k]

        def s_body(s, acc):
            return tuple(acc[j] + buf[s, pl.ds(j * 16, 16)] for j in range(16))

        acc = lax.fori_loop(0, 80, s_body, (zero16,) * 16)
        folded = tuple(acc[j] + acc[j + 8] for j in range(8))
        store_row(b, folded, 1.0 / 160.0)

    ring(NB, tn_fire, tn_wait, tn_consume)
    flush(128)

    roff = 0
    for f in range(12):
        pltpu.sync_copy(pl_Ts[f], tblv.at[pl.ds(roff, PL_VOCABS[f])])
        roff += PL_VOCABS[f]

    roff = 0
    for f in range(12):
        pltpu.sync_copy(pl_is[f].at[pl.ds(base * 20, NB * 20)],
                        idx20.at[pl.ds(0, NB * 20)])
        toff = roff

        def pb_body(b, _, toff=toff):
            def s_body(s, acc):
                row = idx20[pl.ds(b * 20 + s, 16)][0] + toff
                return tuple(acc[j] + tblv[row, pl.ds(j * 16, 16)]
                             for j in range(8))

            acc = lax.fori_loop(0, 20, s_body, (zero16,) * 8)
            store_row(b, acc, 1.0 / 20.0)
            return 0

        lax.fori_loop(0, NB, pb_body, 0)
        flush(256 + f * 128)
        roff += PL_VOCABS[f]

    text128(tc_i, tc_T, 1792)
    pltpu.sync_copy(ag_T.at[pl.ds(0, 1)], t0v)
    text128(ag_i, ag_T, 1920, masked=True)

    cs_bufs = (rcsA, rcsB)

    def cs_fire(j, k):
        pltpu.sync_copy(cs_is[j].at[pl.ds(base, NB)], idx1d.at[k])
        pltpu.async_copy(cs_Ts[j].at[idx1d.at[k]], cs_bufs[k], sems[k])

    def cs_wait(j, k):
        pltpu.make_async_copy(cs_Ts[j].at[pl.ds(0, NB)],
                              cs_bufs[k], sems[k]).wait()

    cs_fire(0, 0)
    for j in range(6):
        k = j % 2
        cs_wait(j, k)
        if j < 5:
            cs_fire(j + 1, 1 - k)
        pltpu.sync_copy(cs_bufs[k],
                        out.at[pl.ds(base, NB), pl.ds(2048 + j * 128, 128)])


def _sc_forward(ps_i, tn_i, tc_i, ag_i, pl_is, cs_is,
                ps_T, tn_T, tc_T, ag_T, pl_Ts, cs_Ts):
    mesh = plsc.VectorSubcoreMesh(core_axis_name="c", subcore_axis_name="s",
                                  num_cores=2, num_subcores=16)
    scratch = [
        pltpu.VMEM((NB * 16,), jnp.int32),
        pltpu.VMEM((NB * 80,), jnp.int32),
        pltpu.VMEM((NB * 20 + 16,), jnp.int32),
        pltpu.VMEM((2, NB), jnp.int32),
        pltpu.VMEM((CH * 16, 128), jnp.float32),
        pltpu.VMEM((CH * 16, 128), jnp.float32),
        pltpu.VMEM((80, 256), jnp.float32),
        pltpu.VMEM((80, 256), jnp.float32),
        pltpu.VMEM((NB, 128), jnp.float32),
        pltpu.VMEM((NB, 128), jnp.float32),
        pltpu.VMEM((TBL_ROWS, 128), jnp.float32),
        pltpu.VMEM((1, 128), jnp.float32),
        pltpu.VMEM((NB, 128), jnp.float32),
        pltpu.SemaphoreType.DMA,
        pltpu.SemaphoreType.DMA,
    ]

    def body(*refs):
        it = list(refs)
        args = (it[0], it[1], it[2], it[3], it[4:16], it[16:22],
                it[22], it[23], it[24], it[25], it[26:38], it[38:44],
                it[44], *it[45:])
        _sc_body(args)

    fn = pl.kernel(body,
                   out_type=jax.ShapeDtypeStruct((B, 22 * D), jnp.float32),
                   mesh=mesh, scratch_types=scratch)
    return fn(ps_i, tn_i, tc_i, ag_i, *pl_is, *cs_is,
              ps_T, tn_T, tc_T, ag_T, *pl_Ts, *cs_Ts)


def kernel(pl_name_src, track_name_pl, track_danceability_pl, track_energy_pl,
           track_key_pl, track_loudness_pl, track_mode_pl, track_speechiness_pl,
           track_acousticness_pl, track_instrumentalness_pl, track_liveness_pl,
           track_valence_pl, track_tempo_pl, time_signature_pl, track_name_can,
           artist_genres_can, track_danceability_can, track_energy_can,
           track_key_can, track_loudness_can, track_mode_can, track_speechiness_can,
           T_pl_name_src, T_track_name_pl, T_track_danceability_pl, T_track_energy_pl,
           T_track_key_pl, T_track_loudness_pl, T_track_mode_pl, T_track_speechiness_pl,
           T_track_acousticness_pl, T_track_instrumentalness_pl, T_track_liveness_pl,
           T_track_valence_pl, T_track_tempo_pl, T_time_signature_pl, T_track_name_can,
           T_artist_genres_can, T_track_danceability_can, T_track_energy_can,
           T_track_key_can, T_track_loudness_can, T_track_mode_can, T_track_speechiness_can):
    pl_is = [track_danceability_pl, track_energy_pl, track_key_pl,
             track_loudness_pl, track_mode_pl, track_speechiness_pl,
             track_acousticness_pl, track_instrumentalness_pl,
             track_liveness_pl, track_valence_pl, track_tempo_pl,
             time_signature_pl]
    cs_is = [track_danceability_can, track_energy_can, track_key_can,
             track_loudness_can, track_mode_can, track_speechiness_can]
    pl_Ts = [T_track_danceability_pl, T_track_energy_pl, T_track_key_pl,
             T_track_loudness_pl, T_track_mode_pl, T_track_speechiness_pl,
             T_track_acousticness_pl, T_track_instrumentalness_pl,
             T_track_liveness_pl, T_track_valence_pl, T_track_tempo_pl,
             T_time_signature_pl]
    cs_Ts = [T_track_danceability_can, T_track_energy_can, T_track_key_can,
             T_track_loudness_can, T_track_mode_can, T_track_speechiness_can]
    pl_is = [jnp.ravel(x) for x in pl_is]
    return _sc_forward(jnp.ravel(pl_name_src), jnp.ravel(track_name_pl),
                       jnp.ravel(track_name_can),
                       jnp.ravel(artist_genres_can), pl_is, cs_is,
                       T_pl_name_src, T_track_name_pl, T_track_name_can,
                       T_artist_genres_can, pl_Ts, cs_Ts)


# speedup vs baseline: 11.7396x; 1.4111x over previous
import jax
import jax.numpy as jnp
from jax import lax
from jax.experimental import pallas as pl
from jax.experimental.pallas import tpu as pltpu
from jax.experimental.pallas import tpu_sc as plsc

B = 1024
D = 128
NW = 32
NB = B // NW
CH = 8
BLK = 128
VPAD = 32

PL_VOCABS = (21, 21, 13, 21, 4, 21, 21, 21, 21, 21, 21, 7)
CS_VOCABS = (21, 21, 13, 21, 4, 21)


def _sc_body(ps_i, tn_i, tc_i, ag_i, ps_T, tn_T, tc_T, ag_T, out,
             idx16, idx80, rsmA, rsmB, rbigA, rbigB, t0v, outv, semA, semB):
    cid = lax.axis_index("c")
    sid = lax.axis_index("s")
    wid = sid * 2 + cid
    base = wid * NB

    zero16 = jnp.zeros((16,), jnp.float32)
    sems = (semA, semB)

    def ring(n, fire, wait, consume):
        fire(0, 0)

        def g_body(g, _):
            c0 = 2 * g
            wait(0)
            fire(c0 + 1, 1)
            consume(c0, 0)
            wait(1)

            @pl.when(c0 + 2 < n)
            def _():
                fire(c0 + 2, 0)

            consume(c0 + 1, 1)
            return 0

        lax.fori_loop(0, n // 2, g_body, 0)

    def store_row(b, vecs, scale):
        for c in range(len(vecs)):
            outv[b, pl.ds(c * 16, 16)] = vecs[c] * scale

    def flush(col):
        pltpu.sync_copy(outv, out.at[pl.ds(base, NB), pl.ds(col, 128)])

    def text128(idx_hbm, T_hbm, col, masked=False):
        S = 16
        bufs = (rsmA, rsmB)
        pltpu.sync_copy(idx_hbm.at[pl.ds(base * S, NB * S)], idx16)

        def fire(c, k):
            pltpu.async_copy(T_hbm.at[idx16.at[pl.ds(c * CH * S, CH * S)]],
                             bufs[k], sems[k])

        def wait(k):
            pltpu.make_async_copy(T_hbm.at[pl.ds(0, CH * S)],
                                  bufs[k], sems[k]).wait()

        def consume(c, k):
            buf = bufs[k]

            def bb_body(bb, _):
                b = c * CH + bb

                def s_body(s, acc):
                    return tuple(acc[j] + buf[bb * S + s, pl.ds(j * 16, 16)]
                                 for j in range(8))

                acc = lax.fori_loop(0, S, s_body, (zero16,) * 8)
                if masked:
                    va = idx16[pl.ds(b * S, 16)]
                    nz = va[0] * 0
                    for s in range(16):
                        nz = nz + jnp.where(va[s] == 0, 1, 0)
                    nzv = lax.broadcast_in_dim(nz.astype(jnp.float32), (16,), ())
                    inv = 1.0 / jnp.maximum(jnp.float32(S) - nzv, 1.0)
                    vecs = tuple((acc[j] - nzv * t0v[0, pl.ds(j * 16, 16)]) * inv
                                 for j in range(8))
                    store_row(b, vecs, 1.0)
                else:
                    store_row(b, acc, 1.0 / S)
                return 0

            lax.fori_loop(0, CH, bb_body, 0)

        ring(NB // CH, fire, wait, consume)
        flush(col)

    text128(ps_i, ps_T, 0)

    pltpu.sync_copy(tn_i.at[pl.ds(base * 80, NB * 80)], idx80)
    tn_bufs = (rbigA, rbigB)

    def tn_fire(b, k):
        pltpu.async_copy(tn_T.at[idx80.at[pl.ds(b * 80, 80)]],
                         tn_bufs[k], sems[k])

    def tn_wait(k):
        pltpu.make_async_copy(tn_T.at[pl.ds(0, 80)], tn_bufs[k], sems[k]).wait()

    def tn_consume(b, k):
        buf = tn_bufs[k]

        def s_body(s, acc):
            return tuple(acc[j] + buf[s, pl.ds(j * 16, 16)] for j in range(16))

        acc = lax.fori_loop(0, 80, s_body, (zero16,) * 16)
        folded = tuple(acc[j] + acc[j + 8] for j in range(8))
        store_row(b, folded, 1.0 / 160.0)

    ring(NB, tn_fire, tn_wait, tn_consume)
    flush(128)

    text128(tc_i, tc_T, 256)
    pltpu.sync_copy(ag_T.at[pl.ds(0, 1)], t0v)
    text128(ag_i, ag_T, 384, masked=True)


def _sc_forward(ps_i, tn_i, tc_i, ag_i, ps_T, tn_T, tc_T, ag_T):
    mesh = plsc.VectorSubcoreMesh(core_axis_name="c", subcore_axis_name="s",
                                  num_cores=2, num_subcores=16)
    scratch = [
        pltpu.VMEM((NB * 16,), jnp.int32),
        pltpu.VMEM((NB * 80,), jnp.int32),
        pltpu.VMEM((CH * 16, 128), jnp.float32),
        pltpu.VMEM((CH * 16, 128), jnp.float32),
        pltpu.VMEM((80, 256), jnp.float32),
        pltpu.VMEM((80, 256), jnp.float32),
        pltpu.VMEM((1, 128), jnp.float32),
        pltpu.VMEM((NB, 128), jnp.float32),
        pltpu.SemaphoreType.DMA,
        pltpu.SemaphoreType.DMA,
    ]
    fn = pl.kernel(_sc_body,
                   out_type=jax.ShapeDtypeStruct((B, 4 * D), jnp.float32),
                   mesh=mesh, scratch_types=scratch)
    return fn(ps_i, tn_i, tc_i, ag_i, ps_T, tn_T, tc_T, ag_T)


def _tc_body(*refs):
    sc_ref = refs[0]
    pl_refs = refs[1:13]
    cs_refs = refs[13:19]
    tbl_ref = refs[19]
    o = refs[20]

    o[:, 0:256] = sc_ref[:, 0:256]
    o[:, 1792:2048] = sc_ref[:, 256:512]

    iota = lax.broadcasted_iota(jnp.int32, (1, VPAD), 1)

    for f in range(12):
        idx = pl_refs[f][...]
        cnt = jnp.zeros((BLK, VPAD), jnp.float32)
        for s in range(20):
            cnt = cnt + (idx[:, s:s + 1] == iota).astype(jnp.float32)
        mm = lax.dot_general(cnt, tbl_ref[pl.ds(f * VPAD, VPAD), :],
                             (((1,), (0,)), ((), ())),
                             preferred_element_type=jnp.float32)
        o[:, 256 + f * 128:256 + (f + 1) * 128] = mm * (1.0 / 20.0)

    for j in range(6):
        oh = (cs_refs[j][...] == iota).astype(jnp.float32)
        mm = lax.dot_general(oh, tbl_ref[pl.ds((12 + j) * VPAD, VPAD), :],
                             (((1,), (0,)), ((), ())),
                             preferred_element_type=jnp.float32)
        o[:, 2048 + j * 128:2048 + (j + 1) * 128] = mm


def _tc_small(sc_out, pl_idx, cs_idx, tbl):
    grid = (B // BLK,)
    in_specs = (
        [pl.BlockSpec((BLK, 4 * D), lambda i: (i, 0))]
        + [pl.BlockSpec((BLK, 20), lambda i: (i, 0))] * 12
        + [pl.BlockSpec((BLK, 1), lambda i: (i, 0))] * 6
        + [pl.BlockSpec((18 * VPAD, 128), lambda i: (0, 0))]
    )
    return pl.pallas_call(
        _tc_body,
        grid=grid,
        in_specs=in_specs,
        out_specs=pl.BlockSpec((BLK, 22 * D), lambda i: (i, 0)),
        out_shape=jax.ShapeDtypeStruct((B, 22 * D), jnp.float32),
    )(sc_out, *pl_idx, *cs_idx, tbl)


def kernel(pl_name_src, track_name_pl, track_danceability_pl, track_energy_pl,
           track_key_pl, track_loudness_pl, track_mode_pl, track_speechiness_pl,
           track_acousticness_pl, track_instrumentalness_pl, track_liveness_pl,
           track_valence_pl, track_tempo_pl, time_signature_pl, track_name_can,
           artist_genres_can, track_danceability_can, track_energy_can,
           track_key_can, track_loudness_can, track_mode_can, track_speechiness_can,
           T_pl_name_src, T_track_name_pl, T_track_danceability_pl, T_track_energy_pl,
           T_track_key_pl, T_track_loudness_pl, T_track_mode_pl, T_track_speechiness_pl,
           T_track_acousticness_pl, T_track_instrumentalness_pl, T_track_liveness_pl,
           T_track_valence_pl, T_track_tempo_pl, T_time_signature_pl, T_track_name_can,
           T_artist_genres_can, T_track_danceability_can, T_track_energy_can,
           T_track_key_can, T_track_loudness_can, T_track_mode_can, T_track_speechiness_can):
    pl_idx = [track_danceability_pl, track_energy_pl, track_key_pl,
              track_loudness_pl, track_mode_pl, track_speechiness_pl,
              track_acousticness_pl, track_instrumentalness_pl,
              track_liveness_pl, track_valence_pl, track_tempo_pl,
              time_signature_pl]
    cs_idx = [track_danceability_can, track_energy_can, track_key_can,
              track_loudness_can, track_mode_can, track_speechiness_can]
    pl_Ts = [T_track_danceability_pl, T_track_energy_pl, T_track_key_pl,
             T_track_loudness_pl, T_track_mode_pl, T_track_speechiness_pl,
             T_track_acousticness_pl, T_track_instrumentalness_pl,
             T_track_liveness_pl, T_track_valence_pl, T_track_tempo_pl,
             T_time_signature_pl]
    cs_Ts = [T_track_danceability_can, T_track_energy_can, T_track_key_can,
             T_track_loudness_can, T_track_mode_can, T_track_speechiness_can]

    sc_out = _sc_forward(jnp.ravel(pl_name_src), jnp.ravel(track_name_pl),
                         jnp.ravel(track_name_can), jnp.ravel(artist_genres_can),
                         T_pl_name_src, T_track_name_pl, T_track_name_can,
                         T_artist_genres_can)

    tbl = jnp.concatenate(
        [jnp.pad(t, ((0, VPAD - t.shape[0]), (0, 0)))
         for t in (pl_Ts + cs_Ts)], axis=0)
    cs_idx = [x[:, None] for x in cs_idx]
    return _tc_small(sc_out, pl_idx, cs_idx, tbl)


# speedup vs baseline: 18.0793x; 1.5400x over previous
import jax
import jax.numpy as jnp
from jax import lax
from jax.experimental import pallas as pl
from jax.experimental.pallas import tpu as pltpu
from jax.experimental.pallas import tpu_sc as plsc

B = 1024
D = 128
NW = 32
NB = B // NW
CH = 8
BLK = 256
VPAD = 32

PL_VOCABS = (21, 21, 13, 21, 4, 21, 21, 21, 21, 21, 21, 7)
CS_VOCABS = (21, 21, 13, 21, 4, 21)


def _sc_body(idx_all, ps_T, tn_T, tc_T, ag_T, out,
             idx16a, idx16b, idx16c, idx80,
             rsmA, rsmB, rbigA, rbigB, t0v, outv, semA, semB, semI):
    cid = lax.axis_index("c")
    sid = lax.axis_index("s")
    wid = sid * 2 + cid
    base = wid * NB

    OFF_PS, OFF_TN, OFF_TC, OFF_AG = 0, B * 16, B * 96, B * 112
    pltpu.async_copy(idx_all.at[pl.ds(OFF_PS + base * 16, NB * 16)], idx16a, semI)
    pltpu.async_copy(idx_all.at[pl.ds(OFF_TN + base * 80, NB * 80)], idx80, semI)
    pltpu.async_copy(idx_all.at[pl.ds(OFF_TC + base * 16, NB * 16)], idx16b, semI)
    pltpu.async_copy(idx_all.at[pl.ds(OFF_AG + base * 16, NB * 16)], idx16c, semI)
    pltpu.make_async_copy(idx_all.at[pl.ds(0, NB * 16)], idx16a, semI).wait()
    pltpu.make_async_copy(idx_all.at[pl.ds(0, NB * 80)], idx80, semI).wait()
    pltpu.make_async_copy(idx_all.at[pl.ds(0, NB * 16)], idx16b, semI).wait()
    pltpu.make_async_copy(idx_all.at[pl.ds(0, NB * 16)], idx16c, semI).wait()

    zero16 = jnp.zeros((16,), jnp.float32)
    sems = (semA, semB)

    def ring(n, fire, wait, consume):
        fire(0, 0)
        fire(1, 1)

        def g_body(g, _):
            c0 = 2 * g
            wait(0)
            consume(c0, 0)

            @pl.when(c0 + 2 < n)
            def _():
                fire(c0 + 2, 0)

            wait(1)
            consume(c0 + 1, 1)

            @pl.when(c0 + 3 < n)
            def _():
                fire(c0 + 3, 1)

            return 0

        lax.fori_loop(0, n // 2, g_body, 0)

    def store_row(b, vecs, scale):
        for c in range(len(vecs)):
            outv[b, pl.ds(c * 16, 16)] = vecs[c] * scale

    def flush(f):
        pltpu.sync_copy(outv, out.at[f, pl.ds(base, NB)])

    def text128(idx16, T_hbm, fslot, masked=False):
        S = 16
        bufs = (rsmA, rsmB)

        def fire(c, k):
            pltpu.async_copy(T_hbm.at[idx16.at[pl.ds(c * CH * S, CH * S)]],
                             bufs[k], sems[k])

        def wait(k):
            pltpu.make_async_copy(T_hbm.at[pl.ds(0, CH * S)],
                                  bufs[k], sems[k]).wait()

        def consume(c, k):
            buf = bufs[k]

            def bb_body(bb, _):
                b = c * CH + bb

                def s_body(s, acc):
                    return tuple(acc[j] + buf[bb * S + s, pl.ds(j * 16, 16)]
                                 for j in range(8))

                acc = lax.fori_loop(0, S, s_body, (zero16,) * 8, unroll=4)
                if masked:
                    va = idx16[pl.ds(b * S, 16)]
                    nz = va[0] * 0
                    for s in range(16):
                        nz = nz + jnp.where(va[s] == 0, 1, 0)
                    nzv = lax.broadcast_in_dim(nz.astype(jnp.float32), (16,), ())
                    inv = 1.0 / jnp.maximum(jnp.float32(S) - nzv, 1.0)
                    vecs = tuple((acc[j] - nzv * t0v[0, pl.ds(j * 16, 16)]) * inv
                                 for j in range(8))
                    store_row(b, vecs, 1.0)
                else:
                    store_row(b, acc, 1.0 / S)
                return 0

            lax.fori_loop(0, CH, bb_body, 0)

        ring(NB // CH, fire, wait, consume)
        flush(fslot)

    text128(idx16a, ps_T, 0)

    tn_bufs = (rbigA, rbigB)

    def tn_fire(b, k):
        pltpu.async_copy(tn_T.at[idx80.at[pl.ds(b * 80, 80)]],
                         tn_bufs[k], sems[k])

    def tn_wait(k):
        pltpu.make_async_copy(tn_T.at[pl.ds(0, 80)], tn_bufs[k], sems[k]).wait()

    def tn_consume(b, k):
        buf = tn_bufs[k]

        def s_body(s, acc):
            return tuple(acc[j] + buf[s, pl.ds(j * 16, 16)]
                         + buf[s, pl.ds(128 + j * 16, 16)] for j in range(8))

        folded = lax.fori_loop(0, 80, s_body, (zero16,) * 8, unroll=4)
        store_row(b, folded, 1.0 / 160.0)

    ring(NB, tn_fire, tn_wait, tn_consume)
    flush(1)

    text128(idx16b, tc_T, 2)
    pltpu.sync_copy(ag_T.at[pl.ds(0, 1)], t0v)
    text128(idx16c, ag_T, 3, masked=True)


def _sc_forward(idx_all, ps_T, tn_T, tc_T, ag_T):
    mesh = plsc.VectorSubcoreMesh(core_axis_name="c", subcore_axis_name="s",
                                  num_cores=2, num_subcores=16)
    scratch = [
        pltpu.VMEM((NB * 16,), jnp.int32),
        pltpu.VMEM((NB * 16,), jnp.int32),
        pltpu.VMEM((NB * 16,), jnp.int32),
        pltpu.VMEM((NB * 80,), jnp.int32),
        pltpu.VMEM((CH * 16, 128), jnp.float32),
        pltpu.VMEM((CH * 16, 128), jnp.float32),
        pltpu.VMEM((80, 256), jnp.float32),
        pltpu.VMEM((80, 256), jnp.float32),
        pltpu.VMEM((1, 128), jnp.float32),
        pltpu.VMEM((NB, 128), jnp.float32),
        pltpu.SemaphoreType.DMA,
        pltpu.SemaphoreType.DMA,
        pltpu.SemaphoreType.DMA,
    ]
    fn = pl.kernel(_sc_body,
                   out_type=jax.ShapeDtypeStruct((4, B, D), jnp.float32),
                   mesh=mesh, scratch_types=scratch)
    return fn(idx_all, ps_T, tn_T, tc_T, ag_T)


def _tc_body(*refs):
    sc_ref = refs[0]
    idxT_ref = refs[1]
    tbl_refs = refs[2:20]
    o = refs[20]

    o[:, 0:128] = sc_ref[0]
    o[:, 128:256] = sc_ref[1]
    o[:, 1792:1920] = sc_ref[2]
    o[:, 1920:2048] = sc_ref[3]

    iotaV = lax.broadcasted_iota(jnp.int32, (VPAD, BLK), 0)

    for f in range(12):
        v = PL_VOCABS[f]
        rows = idxT_ref[pl.ds(f * 20, 20), :]
        cnt = jnp.zeros((VPAD, BLK), jnp.float32)
        for s in range(20):
            cnt = cnt + (rows[s:s + 1, :] == iotaV).astype(jnp.float32)
        mm = lax.dot_general(cnt[0:v, :], tbl_refs[f][...],
                             (((0,), (0,)), ((), ())),
                             preferred_element_type=jnp.float32)
        o[:, 256 + f * 128:256 + (f + 1) * 128] = mm * (1.0 / 20.0)

    for j in range(6):
        v = CS_VOCABS[j]
        oh = (idxT_ref[pl.ds(240 + j, 1), :] == iotaV).astype(jnp.float32)
        mm = lax.dot_general(oh[0:v, :], tbl_refs[12 + j][...],
                             (((0,), (0,)), ((), ())),
                             preferred_element_type=jnp.float32)
        o[:, 2048 + j * 128:2048 + (j + 1) * 128] = mm


def _tc_small(sc_out, idxT, tbls):
    grid = (B // BLK,)
    in_specs = (
        [pl.BlockSpec((4, BLK, D), lambda i: (0, i, 0)),
         pl.BlockSpec((248, BLK), lambda i: (0, i))]
        + [pl.BlockSpec(t.shape, lambda i: (0, 0)) for t in tbls]
    )
    return pl.pallas_call(
        _tc_body,
        grid=grid,
        in_specs=in_specs,
        out_specs=pl.BlockSpec((BLK, 22 * D), lambda i: (i, 0)),
        out_shape=jax.ShapeDtypeStruct((B, 22 * D), jnp.float32),
    )(sc_out, idxT, *tbls)


def kernel(pl_name_src, track_name_pl, track_danceability_pl, track_energy_pl,
           track_key_pl, track_loudness_pl, track_mode_pl, track_speechiness_pl,
           track_acousticness_pl, track_instrumentalness_pl, track_liveness_pl,
           track_valence_pl, track_tempo_pl, time_signature_pl, track_name_can,
           artist_genres_can, track_danceability_can, track_energy_can,
           track_key_can, track_loudness_can, track_mode_can, track_speechiness_can,
           T_pl_name_src, T_track_name_pl, T_track_danceability_pl, T_track_energy_pl,
           T_track_key_pl, T_track_loudness_pl, T_track_mode_pl, T_track_speechiness_pl,
           T_track_acousticness_pl, T_track_instrumentalness_pl, T_track_liveness_pl,
           T_track_valence_pl, T_track_tempo_pl, T_time_signature_pl, T_track_name_can,
           T_artist_genres_can, T_track_danceability_can, T_track_energy_can,
           T_track_key_can, T_track_loudness_can, T_track_mode_can, T_track_speechiness_can):
    pl_idx = [track_danceability_pl, track_energy_pl, track_key_pl,
              track_loudness_pl, track_mode_pl, track_speechiness_pl,
              track_acousticness_pl, track_instrumentalness_pl,
              track_liveness_pl, track_valence_pl, track_tempo_pl,
              time_signature_pl]
    cs_idx = [track_danceability_can, track_energy_can, track_key_can,
              track_loudness_can, track_mode_can, track_speechiness_can]
    pl_Ts = [T_track_danceability_pl, T_track_energy_pl, T_track_key_pl,
             T_track_loudness_pl, T_track_mode_pl, T_track_speechiness_pl,
             T_track_acousticness_pl, T_track_instrumentalness_pl,
             T_track_liveness_pl, T_track_valence_pl, T_track_tempo_pl,
             T_time_signature_pl]
    cs_Ts = [T_track_danceability_can, T_track_energy_can, T_track_key_can,
             T_track_loudness_can, T_track_mode_can, T_track_speechiness_can]

    idx_all = jnp.concatenate(
        [jnp.ravel(pl_name_src), jnp.ravel(track_name_pl),
         jnp.ravel(track_name_can), jnp.ravel(artist_genres_can)])
    sc_out = _sc_forward(idx_all, T_pl_name_src, T_track_name_pl,
                         T_track_name_can, T_artist_genres_can)

    idxT = jnp.concatenate(
        pl_idx + [x[:, None] for x in cs_idx]
        + [cs_idx[0][:, None], cs_idx[0][:, None]], axis=1).T
    return _tc_small(sc_out, idxT, pl_Ts + cs_Ts)


# speedup vs baseline: 18.1384x; 1.0033x over previous
import jax
import jax.numpy as jnp
from jax import lax
from jax.experimental import pallas as pl
from jax.experimental.pallas import tpu as pltpu
from jax.experimental.pallas import tpu_sc as plsc

B = 1024
D = 128
NW = 32
NB = B // NW
CH = 8
BLK = 256
VPAD = 32

PL_VOCABS = (21, 21, 13, 21, 4, 21, 21, 21, 21, 21, 21, 7)
CS_VOCABS = (21, 21, 13, 21, 4, 21)


def _sc_body(ps_i, tn_i, tc_i, ag_i, ps_T, tn_T, tc_T, ag_T, out,
             idx16a, idx16b, idx16c, idx80,
             rsmA, rsmB, rbigA, rbigB, t0v, outv0, outv1,
             semA, semB, semC, semD, semI, semF, semG):
    cid = lax.axis_index("c")
    sid = lax.axis_index("s")
    wid = sid * 2 + cid
    base = wid * NB

    pltpu.async_copy(ps_i.at[pl.ds(base * 16, NB * 16)], idx16a, semI)
    pltpu.async_copy(tn_i.at[pl.ds(base * 80, NB * 80)], idx80, semI)
    pltpu.async_copy(tc_i.at[pl.ds(base * 16, NB * 16)], idx16b, semI)
    pltpu.async_copy(ag_i.at[pl.ds(base * 16, NB * 16)], idx16c, semI)
    pltpu.make_async_copy(ps_i.at[pl.ds(0, NB * 16)], idx16a, semI).wait()
    pltpu.make_async_copy(tn_i.at[pl.ds(0, NB * 80)], idx80, semI).wait()
    pltpu.make_async_copy(tc_i.at[pl.ds(0, NB * 16)], idx16b, semI).wait()
    pltpu.make_async_copy(ag_i.at[pl.ds(0, NB * 16)], idx16c, semI).wait()

    zero16 = jnp.zeros((16,), jnp.float32)

    def ring_loop(n, fire, wait, consume):
        def g_body(g, _):
            c0 = 2 * g
            wait(0)
            consume(c0, 0)

            @pl.when(c0 + 2 < n)
            def _():
                fire(c0 + 2, 0)

            wait(1)
            consume(c0 + 1, 1)

            @pl.when(c0 + 3 < n)
            def _():
                fire(c0 + 3, 1)

            return 0

        lax.fori_loop(0, n // 2, g_body, 0)

    def store_row(outv, b, vecs, scale):
        for c in range(len(vecs)):
            outv[b, pl.ds(c * 16, 16)] = vecs[c] * scale

    def flush(outv, f, sem):
        pltpu.async_copy(outv, out.at[f, pl.ds(base, NB)], sem)

    def drain_flush(sem):
        pltpu.make_async_copy(out.at[0, pl.ds(0, NB)], outv0, sem).wait()

    def make_text(idx16, T_hbm, outv, masked):
        S = 16
        bufs = (rsmA, rsmB)
        sems = (semA, semB)

        def fire(c, k):
            pltpu.async_copy(T_hbm.at[idx16.at[pl.ds(c * CH * S, CH * S)]],
                             bufs[k], sems[k])

        def wait(k):
            pltpu.make_async_copy(T_hbm.at[pl.ds(0, CH * S)],
                                  bufs[k], sems[k]).wait()

        def consume(c, k):
            buf = bufs[k]

            def bb_body(bb, _):
                b = c * CH + bb

                def s_body(s, acc):
                    return tuple(acc[j] + buf[bb * S + s, pl.ds(j * 16, 16)]
                                 for j in range(8))

                acc = lax.fori_loop(0, S, s_body, (zero16,) * 8, unroll=4)
                if masked:
                    va = idx16[pl.ds(b * S, 16)]
                    nz = va[0] * 0
                    for s in range(16):
                        nz = nz + jnp.where(va[s] == 0, 1, 0)
                    nzv = lax.broadcast_in_dim(nz.astype(jnp.float32), (16,), ())
                    inv = 1.0 / jnp.maximum(jnp.float32(S) - nzv, 1.0)
                    vecs = tuple((acc[j] - nzv * t0v[0, pl.ds(j * 16, 16)]) * inv
                                 for j in range(8))
                    store_row(outv, b, vecs, 1.0)
                else:
                    store_row(outv, b, acc, 1.0 / S)
                return 0

            lax.fori_loop(0, CH, bb_body, 0)

        return fire, wait, consume

    tn_bufs = (rbigA, rbigB)
    tn_sems = (semC, semD)

    def tn_fire(b, k):
        pltpu.async_copy(tn_T.at[idx80.at[pl.ds(b * 80, 80)]],
                         tn_bufs[k], tn_sems[k])

    def tn_wait(k):
        pltpu.make_async_copy(tn_T.at[pl.ds(0, 80)],
                              tn_bufs[k], tn_sems[k]).wait()

    def tn_consume(b, k):
        buf = tn_bufs[k]

        def s_body(s, acc):
            return tuple(acc[j] + buf[s, pl.ds(j * 16, 16)]
                         + buf[s, pl.ds(128 + j * 16, 16)] for j in range(8))

        folded = lax.fori_loop(0, 80, s_body, (zero16,) * 8, unroll=4)
        store_row(outv1, b, folded, 1.0 / 160.0)

    ps_fns = make_text(idx16a, ps_T, outv0, False)
    tc_fns = make_text(idx16b, tc_T, outv0, False)
    ag_fns = make_text(idx16c, ag_T, outv1, True)

    tn_fire(0, 0)
    tn_fire(1, 1)

    ps_fns[0](0, 0)
    ps_fns[0](1, 1)
    ring_loop(NB // CH, *ps_fns)
    flush(outv0, 0, semF)

    tc_fns[0](0, 0)
    tc_fns[0](1, 1)

    ring_loop(NB, tn_fire, tn_wait, tn_consume)
    flush(outv1, 1, semG)
    pltpu.sync_copy(ag_T.at[pl.ds(0, 1)], t0v)

    drain_flush(semF)
    ring_loop(NB // CH, *tc_fns)
    flush(outv0, 2, semF)

    ag_fns[0](0, 0)
    ag_fns[0](1, 1)
    drain_flush(semG)
    ring_loop(NB // CH, *ag_fns)
    flush(outv1, 3, semG)

    drain_flush(semF)
    drain_flush(semG)


def _sc_forward(ps_i, tn_i, tc_i, ag_i, ps_T, tn_T, tc_T, ag_T):
    mesh = plsc.VectorSubcoreMesh(core_axis_name="c", subcore_axis_name="s",
                                  num_cores=2, num_subcores=16)
    scratch = [
        pltpu.VMEM((NB * 16,), jnp.int32),
        pltpu.VMEM((NB * 16,), jnp.int32),
        pltpu.VMEM((NB * 16,), jnp.int32),
        pltpu.VMEM((NB * 80,), jnp.int32),
        pltpu.VMEM((CH * 16, 128), jnp.float32),
        pltpu.VMEM((CH * 16, 128), jnp.float32),
        pltpu.VMEM((80, 256), jnp.float32),
        pltpu.VMEM((80, 256), jnp.float32),
        pltpu.VMEM((1, 128), jnp.float32),
        pltpu.VMEM((NB, 128), jnp.float32),
        pltpu.VMEM((NB, 128), jnp.float32),
        pltpu.SemaphoreType.DMA,
        pltpu.SemaphoreType.DMA,
        pltpu.SemaphoreType.DMA,
        pltpu.SemaphoreType.DMA,
        pltpu.SemaphoreType.DMA,
        pltpu.SemaphoreType.DMA,
        pltpu.SemaphoreType.DMA,
    ]
    fn = pl.kernel(_sc_body,
                   out_type=jax.ShapeDtypeStruct((4, B, D), jnp.float32),
                   mesh=mesh, scratch_types=scratch)
    return fn(ps_i, tn_i, tc_i, ag_i, ps_T, tn_T, tc_T, ag_T)


def _tc_body(*refs):
    sc_ref = refs[0]
    idxT_ref = refs[1]
    tbl_refs = refs[2:20]
    o = refs[20]

    o[:, 0:128] = sc_ref[0]
    o[:, 128:256] = sc_ref[1]
    o[:, 1792:1920] = sc_ref[2]
    o[:, 1920:2048] = sc_ref[3]

    iotaV = lax.broadcasted_iota(jnp.int32, (VPAD, BLK), 0)

    for f in range(12):
        v = PL_VOCABS[f]
        rows = idxT_ref[pl.ds(f * 20, 20), :]
        cnt = jnp.zeros((VPAD, BLK), jnp.float32)
        for s in range(20):
            cnt = cnt + (rows[s:s + 1, :] == iotaV).astype(jnp.float32)
        mm = lax.dot_general(cnt[0:v, :], tbl_refs[f][...],
                             (((0,), (0,)), ((), ())),
                             preferred_element_type=jnp.float32)
        o[:, 256 + f * 128:256 + (f + 1) * 128] = mm * (1.0 / 20.0)

    for j in range(6):
        v = CS_VOCABS[j]
        oh = (idxT_ref[pl.ds(240 + j, 1), :] == iotaV).astype(jnp.float32)
        mm = lax.dot_general(oh[0:v, :], tbl_refs[12 + j][...],
                             (((0,), (0,)), ((), ())),
                             preferred_element_type=jnp.float32)
        o[:, 2048 + j * 128:2048 + (j + 1) * 128] = mm


def _tc_small(sc_out, idxT, tbls):
    grid = (B // BLK,)
    in_specs = (
        [pl.BlockSpec((4, BLK, D), lambda i: (0, i, 0)),
         pl.BlockSpec((248, BLK), lambda i: (0, i))]
        + [pl.BlockSpec(t.shape, lambda i: (0, 0)) for t in tbls]
    )
    return pl.pallas_call(
        _tc_body,
        grid=grid,
        in_specs=in_specs,
        out_specs=pl.BlockSpec((BLK, 22 * D), lambda i: (i, 0)),
        out_shape=jax.ShapeDtypeStruct((B, 22 * D), jnp.float32),
    )(sc_out, idxT, *tbls)


def kernel(pl_name_src, track_name_pl, track_danceability_pl, track_energy_pl,
           track_key_pl, track_loudness_pl, track_mode_pl, track_speechiness_pl,
           track_acousticness_pl, track_instrumentalness_pl, track_liveness_pl,
           track_valence_pl, track_tempo_pl, time_signature_pl, track_name_can,
           artist_genres_can, track_danceability_can, track_energy_can,
           track_key_can, track_loudness_can, track_mode_can, track_speechiness_can,
           T_pl_name_src, T_track_name_pl, T_track_danceability_pl, T_track_energy_pl,
           T_track_key_pl, T_track_loudness_pl, T_track_mode_pl, T_track_speechiness_pl,
           T_track_acousticness_pl, T_track_instrumentalness_pl, T_track_liveness_pl,
           T_track_valence_pl, T_track_tempo_pl, T_time_signature_pl, T_track_name_can,
           T_artist_genres_can, T_track_danceability_can, T_track_energy_can,
           T_track_key_can, T_track_loudness_can, T_track_mode_can, T_track_speechiness_can):
    pl_idx = [track_danceability_pl, track_energy_pl, track_key_pl,
              track_loudness_pl, track_mode_pl, track_speechiness_pl,
              track_acousticness_pl, track_instrumentalness_pl,
              track_liveness_pl, track_valence_pl, track_tempo_pl,
              time_signature_pl]
    cs_idx = [track_danceability_can, track_energy_can, track_key_can,
              track_loudness_can, track_mode_can, track_speechiness_can]
    pl_Ts = [T_track_danceability_pl, T_track_energy_pl, T_track_key_pl,
             T_track_loudness_pl, T_track_mode_pl, T_track_speechiness_pl,
             T_track_acousticness_pl, T_track_instrumentalness_pl,
             T_track_liveness_pl, T_track_valence_pl, T_track_tempo_pl,
             T_time_signature_pl]
    cs_Ts = [T_track_danceability_can, T_track_energy_can, T_track_key_can,
             T_track_loudness_can, T_track_mode_can, T_track_speechiness_can]

    sc_out = _sc_forward(jnp.ravel(pl_name_src), jnp.ravel(track_name_pl),
                         jnp.ravel(track_name_can), jnp.ravel(artist_genres_can),
                         T_pl_name_src, T_track_name_pl,
                         T_track_name_can, T_artist_genres_can)

    idxT = jnp.concatenate(
        pl_idx + [x[:, None] for x in cs_idx]
        + [cs_idx[0][:, None], cs_idx[0][:, None]], axis=1).T
    return _tc_small(sc_out, idxT, pl_Ts + cs_Ts)
